# Initial kernel scaffold; baseline (speedup 1.0000x reference)
import jax
import jax.numpy as jnp
from jax.experimental import pallas as pl


def kernel(x, edge_index, W1, b1, W2, b2, W3, b3):
    raise NotImplementedError("write your pallas kernel here")


# speedup vs baseline: 9.6009x; 9.6009x over previous
import functools

import jax
import jax.numpy as jnp
from jax import lax
from jax.experimental import pallas as pl
from jax.experimental.pallas import tpu as pltpu
from jax.experimental.pallas import tpu_sc as plsc

NC = 2
NS = 16
NW = NC * NS
LANES = 16
CHUNK = 128


def _pad_sizes(n_edges, n_nodes):
    c = -(-n_edges // (NW * CHUNK))
    c += c % 2
    e_pad = NW * CHUNK * c
    npad = -(-(n_nodes + 1) // (NS * 16)) * (NS * 16)
    return c, e_pad, npad


def _degree_body(n_pad, e_w, dst_hbm, out_hbm, dst_v, acc):
    c = lax.axis_index("c")
    s = lax.axis_index("s")
    w = c * NS + s
    ones16 = jnp.full((LANES,), 1.0, jnp.float32)
    zero16 = jnp.full((LANES,), 0.0, jnp.float32)
    pltpu.sync_copy(dst_hbm.at[w], dst_v)

    def zbody(i, carry):
        acc[pl.ds(i * LANES, LANES)] = zero16
        return carry

    lax.fori_loop(0, n_pad // LANES, zbody, 0)

    def ebody(i, carry):
        di = dst_v[pl.ds(i * LANES, LANES)]
        plsc.addupdate_scatter(acc, [di], ones16)
        return carry

    lax.fori_loop(0, e_w // LANES, ebody, 0)
    pltpu.sync_copy(acc, out_hbm.at[w])


def _make_degree_kernel(n_pad, e_w):
    mesh = plsc.VectorSubcoreMesh(core_axis_name="c", subcore_axis_name="s")
    body = functools.partial(_degree_body, n_pad, e_w)
    return pl.kernel(
        body,
        out_type=jax.ShapeDtypeStruct((NW, n_pad), jnp.float32),
        mesh=mesh,
        compiler_params=pltpu.CompilerParams(needs_layout_passes=False),
        scratch_types=[
            pltpu.VMEM((e_w,), jnp.int32),
            pltpu.VMEM((n_pad,), jnp.float32),
        ],
    )


def _agg_body(n_pad, n_chunks, hid,
              hs_hbm, src_hbm, dst_hbm, zeros_hbm, out_hbm,
              src_v, dst_v, rows, acc):
    c = lax.axis_index("c")
    s = lax.axis_index("s")
    w = c * NS + s
    rows_per_tile = n_pad // NS
    pltpu.sync_copy(zeros_hbm, acc.at[pl.ds(s * rows_per_tile, rows_per_tile)])
    pltpu.sync_copy(src_hbm.at[w], src_v)
    pltpu.sync_copy(dst_hbm.at[w], dst_v)
    plsc.subcore_barrier()

    def ebody(j, carry):
        pltpu.sync_copy(hs_hbm.at[src_v.at[j]], rows)
        pltpu.sync_copy(rows, acc.at[dst_v.at[j]], add=True)
        return carry

    lax.fori_loop(0, n_chunks, ebody, 0)
    plsc.subcore_barrier()
    pltpu.sync_copy(
        acc.at[pl.ds(s * rows_per_tile, rows_per_tile)],
        out_hbm.at[c, pl.ds(s * rows_per_tile, rows_per_tile)],
    )


def _make_agg_kernel(n_pad, n_chunks, hid):
    mesh = plsc.VectorSubcoreMesh(core_axis_name="c", subcore_axis_name="s")
    body = functools.partial(_agg_body, n_pad, n_chunks, hid)
    return pl.kernel(
        body,
        out_type=jax.ShapeDtypeStruct((NC, n_pad, hid), jnp.float32),
        mesh=mesh,
        compiler_params=pltpu.CompilerParams(needs_layout_passes=False),
        scratch_types=[
            pltpu.VMEM((n_chunks, CHUNK), jnp.int32),
            pltpu.VMEM((n_chunks, CHUNK), jnp.int32),
            pltpu.VMEM((CHUNK, hid), jnp.float32),
            pltpu.VMEM_SHARED((n_pad, hid), jnp.float32),
        ],
    )


def _scalar_agg_body(n_nodes, n_pad, e_w,
                     z_hbm, src_hbm, dst_hbm, out_hbm,
                     z_v, src_v, dst_v, acc):
    zero16 = jnp.full((LANES,), 0.0, jnp.float32)
    c = lax.axis_index("c")
    s = lax.axis_index("s")
    w = c * NS + s
    pltpu.sync_copy(z_hbm, z_v)
    pltpu.sync_copy(src_hbm.at[w], src_v)
    pltpu.sync_copy(dst_hbm.at[w], dst_v)

    def zbody(i, carry):
        acc[pl.ds(i * LANES, LANES)] = zero16
        return carry

    lax.fori_loop(0, n_pad // LANES, zbody, 0)

    def ebody(i, carry):
        si = src_v[pl.ds(i * LANES, LANES)]
        di = dst_v[pl.ds(i * LANES, LANES)]
        vals = plsc.load_gather(z_v, [si])
        plsc.addupdate_scatter(acc, [di], vals)
        return carry

    lax.fori_loop(0, e_w // LANES, ebody, 0)
    pltpu.sync_copy(acc, out_hbm.at[w])


def _make_scalar_agg_kernel(n_nodes, n_pad, e_w):
    mesh = plsc.VectorSubcoreMesh(core_axis_name="c", subcore_axis_name="s")
    body = functools.partial(_scalar_agg_body, n_nodes, n_pad, e_w)
    return pl.kernel(
        body,
        out_type=jax.ShapeDtypeStruct((NW, n_pad), jnp.float32),
        mesh=mesh,
        compiler_params=pltpu.CompilerParams(needs_layout_passes=False),
        scratch_types=[
            pltpu.VMEM((n_nodes,), jnp.float32),
            pltpu.VMEM((e_w,), jnp.int32),
            pltpu.VMEM((e_w,), jnp.int32),
            pltpu.VMEM((n_pad,), jnp.float32),
        ],
    )


def _dinv_from_partials(degp):
    deg = jnp.sum(degp, axis=0) + 1.0
    return lax.rsqrt(deg)


def _tc_first_body(degp_ref, x_ref, w_ref, hs_ref):
    dinv = _dinv_from_partials(degp_ref[...])
    h = jnp.dot(x_ref[...], w_ref[...], preferred_element_type=jnp.float32)
    hs_ref[...] = h * dinv[:, None]


def _tc_mid_body(degp_ref, agg_ref, hsp_ref, b_ref, w_ref, hs_ref):
    dinv = _dinv_from_partials(degp_ref[...])
    a = agg_ref[0] + agg_ref[1] + hsp_ref[...]
    h = jnp.maximum(a * dinv[:, None] + b_ref[...], 0.0)
    hs_ref[...] = (
        jnp.dot(h, w_ref[...], preferred_element_type=jnp.float32)
        * dinv[:, None]
    )


def _tc_final_body(degp_ref, accs_ref, z_ref, b_ref, out_ref):
    dinv = _dinv_from_partials(degp_ref[...])
    a = jnp.sum(accs_ref[...], axis=0) + z_ref[...][:, 0]
    out_ref[...] = (dinv * a)[:, None] + b_ref[...]


def kernel(x, edge_index, W1, b1, W2, b2, W3, b3):
    n_nodes, in_dim = x.shape
    hid = W1.shape[1]
    n_edges = edge_index.shape[1]
    n_chunks, e_pad, n_pad = _pad_sizes(n_edges, n_nodes)
    e_w = e_pad // NW
    pad = e_pad - n_edges

    src = jnp.concatenate([edge_index[0], jnp.zeros((pad,), jnp.int32)])
    dst = jnp.concatenate(
        [edge_index[1], jnp.full((pad,), n_nodes, jnp.int32)])
    src2d = src.reshape(NW, n_chunks, CHUNK)
    dst2d = dst.reshape(NW, n_chunks, CHUNK)
    src1d = src.reshape(NW, e_w)
    dst1d = dst.reshape(NW, e_w)
    zeros_blk = jnp.zeros((n_pad // NS, hid), jnp.float32)

    degree_k = _make_degree_kernel(n_pad, e_w)
    agg_k = _make_agg_kernel(n_pad, n_chunks, hid)
    scalar_k = _make_scalar_agg_kernel(n_nodes, n_pad, e_w)

    degp = degree_k(dst1d)[:, :n_nodes]

    tc_first = pl.pallas_call(
        _tc_first_body,
        out_shape=jax.ShapeDtypeStruct((n_nodes, hid), jnp.float32),
    )
    tc_mid = functools.partial(
        pl.pallas_call, _tc_mid_body)
    tc_final = pl.pallas_call(
        _tc_final_body,
        out_shape=jax.ShapeDtypeStruct((n_nodes, 1), jnp.float32),
    )

    hs1 = tc_first(degp, x, W1)
    agg1 = agg_k(hs1, src2d, dst2d, zeros_blk)[:, :n_nodes, :]
    hs2 = tc_mid(out_shape=jax.ShapeDtypeStruct((n_nodes, hid), jnp.float32))(
        degp, agg1, hs1, b1.reshape(1, hid), W2)
    agg2 = agg_k(hs2, src2d, dst2d, zeros_blk)[:, :n_nodes, :]
    z = tc_mid(out_shape=jax.ShapeDtypeStruct((n_nodes, 1), jnp.float32))(
        degp, agg2, hs2, b2.reshape(1, hid), W3)
    accs = scalar_k(z.reshape(n_nodes), src1d, dst1d)[:, :n_nodes]
    out = tc_final(degp, accs, z, b3.reshape(1, 1))
    return out


# speedup vs baseline: 13.7885x; 1.4362x over previous
import functools

import jax
import jax.numpy as jnp
from jax import lax
from jax.experimental import pallas as pl
from jax.experimental.pallas import tpu as pltpu
from jax.experimental.pallas import tpu_sc as plsc

NC = 2
NS = 16
NW = NC * NS
LANES = 16
CHUNK = 104


def _pad_sizes(n_edges, n_nodes):
    c = -(-n_edges // (NW * CHUNK))
    c += c % 2
    e_pad = NW * CHUNK * c
    m = NS * 8
    npad = -(-(n_nodes + 1) // m) * m
    return c, e_pad, npad


def _degree_body(n_pad, e_w, dst_hbm, out_hbm, dst_v, acc):
    c = lax.axis_index("c")
    s = lax.axis_index("s")
    w = c * NS + s
    ones16 = jnp.full((LANES,), 1.0, jnp.float32)
    zero16 = jnp.full((LANES,), 0.0, jnp.float32)
    pltpu.sync_copy(dst_hbm.at[w], dst_v)

    def zbody(i, carry):
        acc[pl.ds(i * LANES, LANES)] = zero16
        return carry

    lax.fori_loop(0, n_pad // LANES, zbody, 0)

    def ebody(i, carry):
        di = dst_v[pl.ds(i * LANES, LANES)]
        plsc.addupdate_scatter(acc, [di], ones16)
        return carry

    lax.fori_loop(0, e_w // LANES, ebody, 0)
    pltpu.sync_copy(acc, out_hbm.at[w])


def _make_degree_kernel(n_pad, e_w):
    mesh = plsc.VectorSubcoreMesh(core_axis_name="c", subcore_axis_name="s")
    body = functools.partial(_degree_body, n_pad, e_w)
    return pl.kernel(
        body,
        out_type=jax.ShapeDtypeStruct((NW, n_pad), jnp.float32),
        mesh=mesh,
        compiler_params=pltpu.CompilerParams(needs_layout_passes=False),
        scratch_types=[
            pltpu.VMEM((e_w,), jnp.int32),
            pltpu.VMEM((n_pad,), jnp.float32),
        ],
    )


def _agg_body(n_pad, n_chunks, hid,
              hs_hbm, src_hbm, dst_hbm, zeros_hbm, out_hbm,
              src_v, dst_v, rows_a, rows_b, acc, sem_g):
    c = lax.axis_index("c")
    s = lax.axis_index("s")
    w = c * NS + s
    rows_per_tile = n_pad // NS
    pltpu.sync_copy(zeros_hbm,
                    acc.at[pl.ds(s * rows_per_tile, rows_per_tile)])
    pltpu.sync_copy(src_hbm.at[w], src_v)
    pltpu.sync_copy(dst_hbm.at[w], dst_v)
    plsc.subcore_barrier()

    def gidx(j):
        return src_v.at[pl.ds(j * CHUNK, CHUNK)]

    pltpu.async_copy(hs_hbm.at[gidx(0)], rows_a, sem_g)

    def ebody(k, carry):
        j0 = 2 * k
        pltpu.make_async_copy(hs_hbm, rows_a, sem_g).wait()
        pltpu.async_copy(hs_hbm.at[gidx(j0 + 1)], rows_b, sem_g)
        pltpu.sync_copy(rows_a, acc.at[dst_v.at[j0]], add=True)
        pltpu.make_async_copy(hs_hbm, rows_b, sem_g).wait()

        @pl.when(j0 + 2 < n_chunks)
        def _():
            pltpu.async_copy(hs_hbm.at[gidx(j0 + 2)], rows_a, sem_g)

        pltpu.sync_copy(rows_b, acc.at[dst_v.at[j0 + 1]], add=True)
        return carry

    lax.fori_loop(0, n_chunks // 2, ebody, 0)
    plsc.subcore_barrier()
    pltpu.sync_copy(
        acc.at[pl.ds(s * rows_per_tile, rows_per_tile)],
        out_hbm.at[c, pl.ds(s * rows_per_tile, rows_per_tile)],
    )


def _make_agg_kernel(n_pad, n_chunks, hid):
    mesh = plsc.VectorSubcoreMesh(core_axis_name="c", subcore_axis_name="s")
    body = functools.partial(_agg_body, n_pad, n_chunks, hid)
    return pl.kernel(
        body,
        out_type=jax.ShapeDtypeStruct((NC, n_pad, hid), jnp.float32),
        mesh=mesh,
        compiler_params=pltpu.CompilerParams(needs_layout_passes=False),
        scratch_types=[
            pltpu.VMEM((n_chunks * CHUNK,), jnp.int32),
            pltpu.VMEM((n_chunks, CHUNK), jnp.int32),
            pltpu.VMEM((CHUNK, hid), jnp.float32),
            pltpu.VMEM((CHUNK, hid), jnp.float32),
            pltpu.VMEM_SHARED((n_pad, hid), jnp.float32),
            pltpu.SemaphoreType.DMA,
        ],
    )


def _scalar_agg_body(n_nodes, n_pad, e_w,
                     z_hbm, src_hbm, dst_hbm, out_hbm,
                     z_v, src_v, dst_v, acc):
    zero16 = jnp.full((LANES,), 0.0, jnp.float32)
    c = lax.axis_index("c")
    s = lax.axis_index("s")
    w = c * NS + s
    pltpu.sync_copy(z_hbm, z_v)
    pltpu.sync_copy(src_hbm.at[w], src_v)
    pltpu.sync_copy(dst_hbm.at[w], dst_v)

    def zbody(i, carry):
        acc[pl.ds(i * LANES, LANES)] = zero16
        return carry

    lax.fori_loop(0, n_pad // LANES, zbody, 0)

    def ebody(i, carry):
        si = src_v[pl.ds(i * LANES, LANES)]
        di = dst_v[pl.ds(i * LANES, LANES)]
        vals = plsc.load_gather(z_v, [si])
        plsc.addupdate_scatter(acc, [di], vals)
        return carry

    lax.fori_loop(0, e_w // LANES, ebody, 0)
    pltpu.sync_copy(acc, out_hbm.at[w])


def _make_scalar_agg_kernel(n_nodes, n_pad, e_w):
    mesh = plsc.VectorSubcoreMesh(core_axis_name="c", subcore_axis_name="s")
    body = functools.partial(_scalar_agg_body, n_nodes, n_pad, e_w)
    return pl.kernel(
        body,
        out_type=jax.ShapeDtypeStruct((NW, n_pad), jnp.float32),
        mesh=mesh,
        compiler_params=pltpu.CompilerParams(needs_layout_passes=False),
        scratch_types=[
            pltpu.VMEM((n_nodes,), jnp.float32),
            pltpu.VMEM((e_w,), jnp.int32),
            pltpu.VMEM((e_w,), jnp.int32),
            pltpu.VMEM((n_pad,), jnp.float32),
        ],
    )


def _dinv_from_partials(degp):
    deg = jnp.sum(degp, axis=0) + 1.0
    return lax.rsqrt(deg)


def _tc_first_body(degp_ref, x_ref, w_ref, hs_ref):
    dinv = _dinv_from_partials(degp_ref[...])
    h = jnp.dot(x_ref[...], w_ref[...], preferred_element_type=jnp.float32)
    hs_ref[...] = h * dinv[:, None]


def _tc_mid_body(degp_ref, agg_ref, hsp_ref, b_ref, w_ref, hs_ref):
    dinv = _dinv_from_partials(degp_ref[...])
    a = agg_ref[0] + agg_ref[1] + hsp_ref[...]
    h = jnp.maximum(a * dinv[:, None] + b_ref[...], 0.0)
    hs_ref[...] = (
        jnp.dot(h, w_ref[...], preferred_element_type=jnp.float32)
        * dinv[:, None]
    )


def _tc_final_body(degp_ref, accs_ref, z_ref, b_ref, out_ref):
    dinv = _dinv_from_partials(degp_ref[...])
    a = jnp.sum(accs_ref[...], axis=0) + z_ref[...][:, 0]
    out_ref[...] = (dinv * a)[:, None] + b_ref[...]


def kernel(x, edge_index, W1, b1, W2, b2, W3, b3):
    n_nodes, in_dim = x.shape
    hid = W1.shape[1]
    n_edges = edge_index.shape[1]
    n_chunks, e_pad, n_pad = _pad_sizes(n_edges, n_nodes)
    e_w = e_pad // NW
    pad = e_pad - n_edges

    src = jnp.concatenate([edge_index[0], jnp.zeros((pad,), jnp.int32)])
    dst = jnp.concatenate(
        [edge_index[1], jnp.full((pad,), n_nodes, jnp.int32)])
    dst2d = dst.reshape(NW, n_chunks, CHUNK)
    src1d = src.reshape(NW, e_w)
    dst1d = dst.reshape(NW, e_w)
    zeros_blk = jnp.zeros((n_pad // NS, hid), jnp.float32)

    degree_k = _make_degree_kernel(n_pad, e_w)
    agg_k = _make_agg_kernel(n_pad, n_chunks, hid)
    scalar_k = _make_scalar_agg_kernel(n_nodes, n_pad, e_w)

    degp = degree_k(dst1d)[:, :n_nodes]

    tc_first = pl.pallas_call(
        _tc_first_body,
        out_shape=jax.ShapeDtypeStruct((n_nodes, hid), jnp.float32),
    )
    tc_mid = functools.partial(
        pl.pallas_call, _tc_mid_body)
    tc_final = pl.pallas_call(
        _tc_final_body,
        out_shape=jax.ShapeDtypeStruct((n_nodes, 1), jnp.float32),
    )

    hs1 = tc_first(degp, x, W1)
    agg1 = agg_k(hs1, src1d, dst2d, zeros_blk)[:, :n_nodes, :]
    hs2 = tc_mid(out_shape=jax.ShapeDtypeStruct((n_nodes, hid), jnp.float32))(
        degp, agg1, hs1, b1.reshape(1, hid), W2)
    agg2 = agg_k(hs2, src1d, dst2d, zeros_blk)[:, :n_nodes, :]
    z = tc_mid(out_shape=jax.ShapeDtypeStruct((n_nodes, 1), jnp.float32))(
        degp, agg2, hs2, b2.reshape(1, hid), W3)
    accs = scalar_k(z.reshape(n_nodes), src1d, dst1d)[:, :n_nodes]
    out = tc_final(degp, accs, z, b3.reshape(1, 1))
    return out
